# baseline (device time: 122913 ns/iter reference)
import jax
import jax.numpy as jnp
from jax import lax
from jax.experimental import pallas as pl
from jax.experimental.pallas import tpu as pltpu

N_DEV = 4


def kernel(ids, E):
    v_shard, d = E.shape
    t = ids.shape[0]

    my = lax.axis_index("i")
    offset = my * v_shard
    local = ids - offset
    in_range = (local >= 0) & (local < v_shard)
    safe = jnp.where(in_range, local, 0)
    rows = jnp.take(E, safe, axis=0)
    partial = jnp.where(in_range[:, None], rows, 0.0).astype(jnp.bfloat16)

    def body(x_ref, out_ref, comm_ref, send_sems, recv_sems):
        my_pos = lax.axis_index("i")
        left = (my_pos - 1) % N_DEV
        right = (my_pos + 1) % N_DEV

        barrier_sem = pltpu.get_barrier_semaphore()
        for nbr in [left, right]:
            pl.semaphore_signal(
                barrier_sem, inc=1,
                device_id=(nbr,), device_id_type=pl.DeviceIdType.MESH,
            )
        pl.semaphore_wait(barrier_sem, 2)

        comm_ref[0] = x_ref[:, :]
        acc = x_ref[:, :].astype(jnp.float32)

        for h in range(N_DEV - 1):
            rdma = pltpu.make_async_remote_copy(
                src_ref=comm_ref.at[h],
                dst_ref=comm_ref.at[h + 1],
                send_sem=send_sems.at[h],
                recv_sem=recv_sems.at[h],
                device_id=(right,),
                device_id_type=pl.DeviceIdType.MESH,
            )
            rdma.start()
            rdma.wait()
            acc = acc + comm_ref[h + 1].astype(jnp.float32)

        out_ref[:, :] = acc

    return pl.pallas_call(
        body,
        out_shape=jax.ShapeDtypeStruct((t, d), jnp.float32),
        in_specs=[pl.BlockSpec(memory_space=pltpu.VMEM)],
        out_specs=pl.BlockSpec(memory_space=pltpu.VMEM),
        scratch_shapes=[
            pltpu.VMEM((N_DEV, t, d), jnp.bfloat16),
            pltpu.SemaphoreType.DMA((N_DEV - 1,)),
            pltpu.SemaphoreType.DMA((N_DEV - 1,)),
        ],
        compiler_params=pltpu.CompilerParams(collective_id=0),
    )(partial)


# device time: 73117 ns/iter; 1.6810x vs baseline; 1.6810x over previous
import jax
import jax.numpy as jnp
from jax import lax
from jax.experimental import pallas as pl
from jax.experimental.pallas import tpu as pltpu

N_DEV = 4


def kernel(ids, E):
    v_shard, d = E.shape
    t = ids.shape[0]
    h = d // 2

    my = lax.axis_index("i")
    offset = my * v_shard
    local = ids - offset
    in_range = (local >= 0) & (local < v_shard)
    safe = jnp.where(in_range, local, 0)
    rows = jnp.take(E, safe, axis=0)
    partial = jnp.where(in_range[:, None], rows, 0.0).astype(jnp.bfloat16)

    def body(x_ref, out_ref, acc_a, acc_b, r1r_a, r1r_b, r2r_a, r2r_b,
             send_sems, recv_sems):
        me = lax.axis_index("i")
        p1 = 3 - me
        p2 = me ^ 1

        half_a = (me // 2) * 512
        other_half_a = 512 - half_a
        q_a = me * 256
        q_a_p2 = p2 * 256

        half_b = jnp.where((me == 1) | (me == 2), 512, 0)
        other_half_b = 512 - half_b
        q_b = half_b + (me // 2) * 256
        q_b_p1 = half_b + (p1 // 2) * 256

        barrier_sem = pltpu.get_barrier_semaphore()
        for nbr in [p1, p2]:
            pl.semaphore_signal(
                barrier_sem, inc=1,
                device_id=(nbr,), device_id_type=pl.DeviceIdType.MESH,
            )
        pl.semaphore_wait(barrier_sem, 2)

        acc_a[:, :] = x_ref[:, :h]
        acc_b[:, :] = x_ref[:, h:]

        def xchg(i, src, dst, dev):
            return pltpu.make_async_remote_copy(
                src_ref=src, dst_ref=dst,
                send_sem=send_sems.at[i], recv_sem=recv_sems.at[i],
                device_id=(dev,), device_id_type=pl.DeviceIdType.MESH,
            )

        r1a = xchg(0, acc_a.at[pl.ds(other_half_a, 512)], r1r_a, p1)
        r1b = xchg(1, acc_b.at[pl.ds(other_half_b, 512)], r1r_b, p2)
        r1a.start()
        r1b.start()
        r1a.wait()
        acc_a[pl.ds(half_a, 512), :] = acc_a[pl.ds(half_a, 512), :] + r1r_a[:, :]
        r1b.wait()
        acc_b[pl.ds(half_b, 512), :] = acc_b[pl.ds(half_b, 512), :] + r1r_b[:, :]

        r2a = xchg(2, acc_a.at[pl.ds(q_a_p2, 256)], r2r_a, p2)
        r2b = xchg(3, acc_b.at[pl.ds(q_b_p1, 256)], r2r_b, p1)
        r2a.start()
        r2b.start()
        r2a.wait()
        acc_a[pl.ds(q_a, 256), :] = acc_a[pl.ds(q_a, 256), :] + r2r_a[:, :]
        r2b.wait()
        acc_b[pl.ds(q_b, 256), :] = acc_b[pl.ds(q_b, 256), :] + r2r_b[:, :]

        ag1a = xchg(4, acc_a.at[pl.ds(q_a, 256)], acc_a.at[pl.ds(q_a, 256)], p2)
        ag1b = xchg(5, acc_b.at[pl.ds(q_b, 256)], acc_b.at[pl.ds(q_b, 256)], p1)
        ag1a.start()
        ag1b.start()
        ag1a.wait()
        ag1b.wait()

        ag2a = xchg(6, acc_a.at[pl.ds(half_a, 512)],
                    acc_a.at[pl.ds(half_a, 512)], p1)
        ag2b = xchg(7, acc_b.at[pl.ds(half_b, 512)],
                    acc_b.at[pl.ds(half_b, 512)], p2)
        ag2a.start()
        ag2b.start()
        ag2a.wait()
        ag2b.wait()

        out_ref[:, :h] = acc_a[:, :].astype(jnp.float32)
        out_ref[:, h:] = acc_b[:, :].astype(jnp.float32)

    return pl.pallas_call(
        body,
        out_shape=jax.ShapeDtypeStruct((t, d), jnp.float32),
        in_specs=[pl.BlockSpec(memory_space=pltpu.VMEM)],
        out_specs=pl.BlockSpec(memory_space=pltpu.VMEM),
        scratch_shapes=[
            pltpu.VMEM((t, h), jnp.bfloat16),
            pltpu.VMEM((t, h), jnp.bfloat16),
            pltpu.VMEM((512, h), jnp.bfloat16),
            pltpu.VMEM((512, h), jnp.bfloat16),
            pltpu.VMEM((256, h), jnp.bfloat16),
            pltpu.VMEM((256, h), jnp.bfloat16),
            pltpu.SemaphoreType.DMA((8,)),
            pltpu.SemaphoreType.DMA((8,)),
        ],
        compiler_params=pltpu.CompilerParams(collective_id=0),
    )(partial)


# device time: 51783 ns/iter; 2.3736x vs baseline; 1.4120x over previous
import jax
import jax.numpy as jnp
from jax import lax
from jax.experimental import pallas as pl
from jax.experimental.pallas import tpu as pltpu

N_DEV = 4
N_RG = 2


def kernel(ids, E):
    v_shard, d = E.shape
    t = ids.shape[0]
    h = d // 2
    rows = t // N_RG
    hr = rows // 2
    qr = rows // 4

    my = lax.axis_index("i")
    local_ids = (ids - my * v_shard).reshape(t, 1)

    n_chunks = 8
    v_chunk = v_shard // n_chunks

    def body(lid_ref, e_ref, out_ref, acc_a, acc_b, e_buf, r1r_a, r1r_b,
             r2r_a, r2r_b, e_sems, send_sems, recv_sems):
        me = lax.axis_index("i")
        p1 = 3 - me
        p2 = me ^ 1

        barrier_sem = pltpu.get_barrier_semaphore()
        for nbr in [p1, p2]:
            pl.semaphore_signal(
                barrier_sem, inc=1,
                device_id=(nbr,), device_id_type=pl.DeviceIdType.MESH,
            )

        e_copy = [
            pltpu.make_async_copy(
                e_ref.at[pl.ds((k % n_chunks) * v_chunk, v_chunk), :],
                e_buf.at[k % 2],
                e_sems.at[k % 2],
            )
            for k in range(2 * n_chunks)
        ]
        e_copy[0].start()

        def matmul_rg(rg):
            base = rg * rows
            lid = lid_ref[pl.ds(base, rows), :]
            for c in range(n_chunks):
                k = rg * n_chunks + c
                if k + 1 < 2 * n_chunks:
                    e_copy[k + 1].start()
                e_copy[k].wait()
                iota = (lax.broadcasted_iota(jnp.int32, (rows, v_chunk), 1)
                        + c * v_chunk)
                oh = (lid == iota).astype(jnp.float32)
                pm = jnp.dot(oh, e_buf[k % 2],
                             preferred_element_type=jnp.float32)
                if c == 0:
                    out_ref[pl.ds(base, rows), :] = pm
                else:
                    out_ref[pl.ds(base, rows), :] = (
                        out_ref[pl.ds(base, rows), :] + pm)
            acc_a[pl.ds(base, rows), :] = out_ref[
                pl.ds(base, rows), :h].astype(jnp.bfloat16)
            acc_b[pl.ds(base, rows), :] = out_ref[
                pl.ds(base, rows), h:].astype(jnp.bfloat16)

        class RG:

            def __init__(self, rg):
                self.rg = rg
                base = rg * rows
                self.half_a = base + (me // 2) * hr
                self.other_half_a = base + hr - (me // 2) * hr
                self.q_a = base + me * qr
                self.q_a_p2 = base + p2 * qr
                hb = jnp.where((me == 1) | (me == 2), hr, 0)
                self.half_b = base + hb
                self.other_half_b = base + hr - hb
                self.q_b = base + hb + (me // 2) * qr
                self.q_b_p1 = base + hb + (p1 // 2) * qr
                s = rg * 8

                def xchg(i, src, dst, dev):
                    return pltpu.make_async_remote_copy(
                        src_ref=src, dst_ref=dst,
                        send_sem=send_sems.at[s + i],
                        recv_sem=recv_sems.at[s + i],
                        device_id=(dev,),
                        device_id_type=pl.DeviceIdType.MESH,
                    )

                self.r1a = xchg(0, acc_a.at[pl.ds(self.other_half_a, hr)],
                                r1r_a.at[rg], p1)
                self.r1b = xchg(1, acc_b.at[pl.ds(self.other_half_b, hr)],
                                r1r_b.at[rg], p2)
                self.r2a = xchg(2, acc_a.at[pl.ds(self.q_a_p2, qr)],
                                r2r_a.at[rg], p2)
                self.r2b = xchg(3, acc_b.at[pl.ds(self.q_b_p1, qr)],
                                r2r_b.at[rg], p1)
                self.ag1a = xchg(4, acc_a.at[pl.ds(self.q_a, qr)],
                                 acc_a.at[pl.ds(self.q_a, qr)], p2)
                self.ag1b = xchg(5, acc_b.at[pl.ds(self.q_b, qr)],
                                 acc_b.at[pl.ds(self.q_b, qr)], p1)
                self.ag2a = xchg(6, acc_a.at[pl.ds(self.half_a, hr)],
                                 acc_a.at[pl.ds(self.half_a, hr)], p1)
                self.ag2b = xchg(7, acc_b.at[pl.ds(self.half_b, hr)],
                                 acc_b.at[pl.ds(self.half_b, hr)], p2)
                self.all_ops = [self.r1a, self.r1b, self.r2a, self.r2b,
                                self.ag1a, self.ag1b, self.ag2a, self.ag2b]

            def r1_start(self):
                self.r1a.start()
                self.r1b.start()

            def r1_finish_r2_start(self):
                self.r1a.wait_recv()
                acc_a[pl.ds(self.half_a, hr), :] = (
                    acc_a[pl.ds(self.half_a, hr), :] + r1r_a[self.rg])
                self.r1b.wait_recv()
                acc_b[pl.ds(self.half_b, hr), :] = (
                    acc_b[pl.ds(self.half_b, hr), :] + r1r_b[self.rg])
                self.r2a.start()
                self.r2b.start()

            def r2_finish_ag1_start(self):
                self.r2a.wait_recv()
                acc_a[pl.ds(self.q_a, qr), :] = (
                    acc_a[pl.ds(self.q_a, qr), :] + r2r_a[self.rg])
                self.r2b.wait_recv()
                acc_b[pl.ds(self.q_b, qr), :] = (
                    acc_b[pl.ds(self.q_b, qr), :] + r2r_b[self.rg])
                self.ag1a.start()
                self.ag1b.start()

            def ag1_finish_ag2_start(self):
                self.ag1a.wait_recv()
                self.ag1b.wait_recv()
                self.ag2a.start()
                self.ag2b.start()
                out_ref[pl.ds(self.half_a, hr), :h] = acc_a[
                    pl.ds(self.half_a, hr), :].astype(jnp.float32)
                out_ref[pl.ds(self.half_b, hr), h:] = acc_b[
                    pl.ds(self.half_b, hr), :].astype(jnp.float32)

            def ag2_finish(self):
                self.ag2a.wait_recv()
                out_ref[pl.ds(self.other_half_a, hr), :h] = acc_a[
                    pl.ds(self.other_half_a, hr), :].astype(jnp.float32)
                self.ag2b.wait_recv()
                out_ref[pl.ds(self.other_half_b, hr), h:] = acc_b[
                    pl.ds(self.other_half_b, hr), :].astype(jnp.float32)

        matmul_rg(0)
        rg0 = RG(0)
        rg1 = RG(1)
        pl.semaphore_wait(barrier_sem, 2)
        rg0.r1_start()
        matmul_rg(1)
        rg0.r1_finish_r2_start()
        rg1.r1_start()
        rg0.r2_finish_ag1_start()
        rg1.r1_finish_r2_start()
        rg0.ag1_finish_ag2_start()
        rg1.r2_finish_ag1_start()
        rg0.ag2_finish()
        rg1.ag1_finish_ag2_start()
        rg1.ag2_finish()
        for op in rg0.all_ops + rg1.all_ops:
            op.wait_send()

    return pl.pallas_call(
        body,
        out_shape=jax.ShapeDtypeStruct((t, d), jnp.float32),
        in_specs=[
            pl.BlockSpec(memory_space=pltpu.VMEM),
            pl.BlockSpec(memory_space=pl.ANY),
        ],
        out_specs=pl.BlockSpec(memory_space=pltpu.VMEM),
        scratch_shapes=[
            pltpu.VMEM((t, h), jnp.bfloat16),
            pltpu.VMEM((t, h), jnp.bfloat16),
            pltpu.VMEM((2, v_chunk, d), jnp.float32),
            pltpu.VMEM((N_RG, hr, h), jnp.bfloat16),
            pltpu.VMEM((N_RG, hr, h), jnp.bfloat16),
            pltpu.VMEM((N_RG, qr, h), jnp.bfloat16),
            pltpu.VMEM((N_RG, qr, h), jnp.bfloat16),
            pltpu.SemaphoreType.DMA((2,)),
            pltpu.SemaphoreType.DMA((16,)),
            pltpu.SemaphoreType.DMA((16,)),
        ],
        compiler_params=pltpu.CompilerParams(collective_id=0),
    )(local_ids, E)
